# baseline (speedup 1.0000x reference)
import numpy as np
import jax
import jax.numpy as jnp
from jax.experimental import pallas as pl
from jax.experimental.pallas import tpu as pltpu
from jax.experimental.pallas import tpu_sc as plsc

_B = 512
_TB = 16384


def _sc_gather(table, idx, mesh):
    h, batch = idx.shape
    vocab, dim = table.shape

    @pl.kernel(
        out_type=jax.ShapeDtypeStruct((h, batch, dim), table.dtype),
        mesh=mesh,
        compiler_params=pltpu.CompilerParams(use_tc_tiling_on_sc=False),
    )
    def gather_kernel(table_hbm, i_hbm, o_hbm):
        def body(i_vmem, o_vmem):
            pltpu.sync_copy(table_hbm.at[i_vmem.at[0]], o_vmem.at[0])

        pltpu.emit_pipeline(
            body,
            grid=(h, batch // _B),
            in_specs=[pl.BlockSpec((1, _B), index_map=lambda i, b: (i, b))],
            out_specs=[
                pl.BlockSpec((1, _B, dim), index_map=lambda i, b: (i, b, 0))
            ],
            core_axis_name=("c", "s"),
            dimension_semantics=(pltpu.PARALLEL, pltpu.PARALLEL),
        )(i_hbm, o_hbm)

    return gather_kernel(table, idx)


def kernel(x, table):
    batch, hist = x.shape
    vocab, dim = table.shape
    xt = x.T

    xt_s = (
        xt.reshape(hist, batch // _TB, 2, _TB // 2)
        .transpose(0, 1, 3, 2)
        .reshape(hist, batch)
    )

    mesh = plsc.VectorSubcoreMesh(core_axis_name="c", subcore_axis_name="s")
    n_chunks = 4
    hc = hist // n_chunks

    gs = [
        _sc_gather(table, xt_s[i * hc : (i + 1) * hc], mesh)
        for i in range(n_chunks)
    ]
    g128s = [g.reshape(hc, batch // 2, 2 * dim) for g in gs]

    out_shape = jax.ShapeDtypeStruct((hist, dim, batch), table.dtype)
    in_block = pl.BlockSpec(
        (1, _TB // 2, 2 * dim), index_map=lambda h: (h, 0, 0)
    )

    out_t = None
    for i, g128 in enumerate(g128s):
        h0 = i * hc

        def body(*refs, _first=(i == 0)):
            in_ref, out_ref = refs[-2], refs[-1]
            vt = in_ref[0].T
            out_ref[0] = jnp.concatenate([vt[:dim], vt[dim:]], axis=1)

        out_spec = pl.BlockSpec(
            (1, dim, _TB), index_map=lambda h, _h0=h0: (h + _h0, 0, 0)
        )
        if out_t is None:
            out_t = pl.pallas_call(
                body,
                grid=(hc,),
                in_specs=[in_block],
                out_specs=out_spec,
                out_shape=out_shape,
            )(g128)
        else:
            out_t = pl.pallas_call(
                body,
                grid=(hc,),
                in_specs=[pl.BlockSpec(memory_space=pl.ANY), in_block],
                out_specs=out_spec,
                out_shape=out_shape,
                input_output_aliases={0: 0},
            )(out_t, g128)

    return out_t.transpose(2, 0, 1)


# speedup vs baseline: 1.1179x; 1.1179x over previous
import numpy as np
import jax
import jax.numpy as jnp
from jax.experimental import pallas as pl
from jax.experimental.pallas import tpu as pltpu
from jax.experimental.pallas import tpu_sc as plsc

_B = 512
_TB = 16384


def _sc_gather(table, idx, mesh):
    h, batch = idx.shape
    vocab, dim = table.shape

    @pl.kernel(
        out_type=jax.ShapeDtypeStruct((h, batch, dim), table.dtype),
        mesh=mesh,
        compiler_params=pltpu.CompilerParams(use_tc_tiling_on_sc=False),
    )
    def gather_kernel(table_hbm, i_hbm, o_hbm):
        def body(i_vmem, o_vmem):
            pltpu.sync_copy(table_hbm.at[i_vmem.at[0]], o_vmem.at[0])

        pltpu.emit_pipeline(
            body,
            grid=(h, batch // _B),
            in_specs=[pl.BlockSpec((1, _B), index_map=lambda i, b: (i, b))],
            out_specs=[
                pl.BlockSpec((1, _B, dim), index_map=lambda i, b: (i, b, 0))
            ],
            core_axis_name=("c", "s"),
            dimension_semantics=(pltpu.PARALLEL, pltpu.PARALLEL),
        )(i_hbm, o_hbm)

    return gather_kernel(table, idx)


def kernel(x, table):
    batch, hist = x.shape
    vocab, dim = table.shape
    xt = x.T

    xt_s = (
        xt.reshape(hist, batch // _TB, 2, _TB // 2)
        .transpose(0, 1, 3, 2)
        .reshape(hist, batch)
    )

    mesh = plsc.VectorSubcoreMesh(core_axis_name="c", subcore_axis_name="s")
    n_chunks = 1
    hc = hist // n_chunks

    gs = [
        _sc_gather(table, xt_s[i * hc : (i + 1) * hc], mesh)
        for i in range(n_chunks)
    ]
    g128s = [g.reshape(hc, batch // 2, 2 * dim) for g in gs]

    out_shape = jax.ShapeDtypeStruct((hist, dim, batch), table.dtype)
    in_block = pl.BlockSpec(
        (1, _TB // 2, 2 * dim), index_map=lambda h: (h, 0, 0)
    )

    out_t = None
    for i, g128 in enumerate(g128s):
        h0 = i * hc

        def body(*refs, _first=(i == 0)):
            in_ref, out_ref = refs[-2], refs[-1]
            vt = in_ref[0].T
            out_ref[0] = jnp.concatenate([vt[:dim], vt[dim:]], axis=1)

        out_spec = pl.BlockSpec(
            (1, dim, _TB), index_map=lambda h, _h0=h0: (h + _h0, 0, 0)
        )
        if out_t is None:
            out_t = pl.pallas_call(
                body,
                grid=(hc,),
                in_specs=[in_block],
                out_specs=out_spec,
                out_shape=out_shape,
            )(g128)
        else:
            out_t = pl.pallas_call(
                body,
                grid=(hc,),
                in_specs=[pl.BlockSpec(memory_space=pl.ANY), in_block],
                out_specs=out_spec,
                out_shape=out_shape,
                input_output_aliases={0: 0},
            )(out_t, g128)

    return out_t.transpose(2, 0, 1)
